# baseline (speedup 1.0000x reference)
import functools

import jax
import jax.numpy as jnp
from jax import lax
from jax.experimental import pallas as pl
from jax.experimental.pallas import tpu as pltpu
from jax.experimental.pallas import tpu_sc as plsc

N = 10000
E = 320000
DF = 128
DE = 16

NC = 2
NS = 16
NW = NC * NS
EPW = E // NW
CE = 400
NCH = EPW // CE
SPC = CE // 16
FPAD = 112
NPAD = 10240
RPS = NPAD // NS
ZG = RPS // 16
PB = 128
PGRP = RPS // PB


def _sc_scatter(feat, recv, iden, with_feat):
    mesh = plsc.VectorSubcoreMesh(core_axis_name="c", subcore_axis_name="s")

    @functools.partial(
        pl.kernel,
        mesh=mesh,
        out_type=jax.ShapeDtypeStruct((NC, NPAD, DE), jnp.float32),
        scratch_types=[
            pltpu.VMEM((CE + FPAD, DE), jnp.float32),
            pltpu.VMEM((SPC, 128), jnp.int32),
            pltpu.VMEM((ZG, 128), jnp.int32),
            pltpu.VMEM((PB + FPAD, DE), jnp.float32),
            pltpu.VMEM_SHARED((NPAD, DE), jnp.float32),
        ],
    )
    def k(feat_hbm, recv_hbm, iden_hbm, out_s, fbuf, idxc, idz, pbuf, acc_sh):
        cid = lax.axis_index("c")
        sid = lax.axis_index("s")
        wid = cid * NS + sid

        def zf(i, _):
            pbuf[i, :] = jnp.zeros((DE,), jnp.float32)
            return 0
        lax.fori_loop(0, PB + FPAD, zf, 0)

        if not with_feat:
            def of(i, _):
                fbuf[i, :] = jnp.ones((DE,), jnp.float32)
                return 0
            lax.fori_loop(0, CE + FPAD, of, 0)

        pltpu.sync_copy(iden_hbm.at[sid], idz)
        for z in range(ZG):
            pltpu.sync_copy(pbuf.at[pl.ds(0, 128)], acc_sh.at[idz.at[z]])
        plsc.subcore_barrier()

        def chunk(c, _):
            pltpu.sync_copy(recv_hbm.at[wid, c], idxc)
            if with_feat:
                pltpu.sync_copy(feat_hbm.at[wid, c],
                                fbuf.at[pl.ds(0, CE)])

            def scat(k2, _):
                pltpu.sync_copy(fbuf.at[pl.ds(k2 * 16, 128)],
                                acc_sh.at[idxc.at[k2]], add=True)
                return 0
            lax.fori_loop(0, SPC, scat, 0)
            return 0
        lax.fori_loop(0, NCH, chunk, 0)

        plsc.subcore_barrier()

        for g in range(PGRP):
            for z in range(PB // 16):
                pltpu.sync_copy(acc_sh.at[idz.at[g * (PB // 16) + z]],
                                pbuf.at[pl.ds(z * 16, 128)])
            pltpu.sync_copy(
                pbuf.at[pl.ds(0, PB)],
                out_s.at[cid, pl.ds(sid * RPS + g * PB, PB)])

    return k(feat, recv, iden)


BLK = 1000


def _tc_body(nf_ref, p_ref, c_ref, w1a_ref, w1b_ref, b1_ref, w2_ref, b2_ref,
             g_ref, bb_ref, out_ref):
    sums = p_ref[0] + p_ref[1]
    cnt = c_ref[0] + c_ref[1]
    agg = sums / jnp.maximum(cnt, 1.0)
    nf = nf_ref[...]
    h = jnp.dot(nf, w1a_ref[...], preferred_element_type=jnp.float32)
    h = h + jnp.dot(agg, w1b_ref[...], preferred_element_type=jnp.float32)
    h = jnp.maximum(h + b1_ref[...], 0.0)
    h = jnp.dot(h, w2_ref[...], preferred_element_type=jnp.float32) + b2_ref[...]
    mu = jnp.mean(h, axis=-1, keepdims=True)
    var = jnp.mean((h - mu) * (h - mu), axis=-1, keepdims=True)
    out_ref[...] = nf + (h - mu) * lax.rsqrt(var + 1e-5) * g_ref[...] + bb_ref[...]


def _tc_update(nf, sums, cnts, W1a, W1b, b1, W2, b2, ln_g, ln_b):
    grid = N // BLK
    return pl.pallas_call(
        _tc_body,
        grid=(grid,),
        in_specs=[
            pl.BlockSpec((BLK, DF), lambda i: (i, 0)),
            pl.BlockSpec((NC, BLK, DE), lambda i: (0, i, 0)),
            pl.BlockSpec((NC, BLK, DE), lambda i: (0, i, 0)),
            pl.BlockSpec((DF, DF), lambda i: (0, 0)),
            pl.BlockSpec((DE, DF), lambda i: (0, 0)),
            pl.BlockSpec((1, DF), lambda i: (0, 0)),
            pl.BlockSpec((DF, DF), lambda i: (0, 0)),
            pl.BlockSpec((1, DF), lambda i: (0, 0)),
            pl.BlockSpec((1, DF), lambda i: (0, 0)),
            pl.BlockSpec((1, DF), lambda i: (0, 0)),
        ],
        out_specs=pl.BlockSpec((BLK, DF), lambda i: (i, 0)),
        out_shape=jax.ShapeDtypeStruct((N, DF), jnp.float32),
    )(nf, sums, cnts, W1a, W1b, b1, W2, b2, ln_g, ln_b)


@jax.jit
def kernel(node_features, edge_features, receivers, W1, b1, W2, b2, ln_g, ln_b):
    feat = edge_features.reshape(NW, NCH, CE, DE)
    recv = jnp.pad(receivers.reshape(NW, NCH, SPC, 16)[..., None],
                   ((0, 0), (0, 0), (0, 0), (0, 0), (0, 7)),
                   constant_values=NPAD - 1).reshape(NW, NCH, SPC, 128)
    iden = jnp.pad(jnp.arange(NPAD, dtype=jnp.int32
                              ).reshape(NS, ZG, 16)[..., None],
                   ((0, 0), (0, 0), (0, 0), (0, 7)),
                   constant_values=NPAD - 1).reshape(NS, ZG, 128)
    sums = _sc_scatter(feat, recv, iden, True)
    guard = (sums[0, NPAD - 2, 0] * 0.0).astype(jnp.int32)
    cnts = _sc_scatter(feat, recv, iden + guard, False)

    nf = node_features.reshape(N, DF)
    out = _tc_update(
        nf, sums, cnts,
        W1[:DF], W1[DF:],
        b1.reshape(1, DF), W2, b2.reshape(1, DF),
        ln_g.reshape(1, DF), ln_b.reshape(1, DF),
    )
    return out.reshape(1, N, DF)
